# baseline (speedup 1.0000x reference)
import jax
import jax.numpy as jnp
from jax import lax
from jax.experimental import pallas as pl
from jax.experimental.pallas import tpu as pltpu
from jax.experimental.pallas import tpu_sc as plsc

N = 10000
D = 128
NP = 10240
NC = 2
NS = 16
LANES = 16
NW = NC * NS
BB = 32
E0 = 320000
WB = 320
CHB = 8
NCH = WB // CHB
EP = NW * WB * BB
NT = 10112
DUM = N
RPT = NT // NS
RB = 1024


def _edge_body(h_hbm, as_hbm, ad_hbm, src_hbm, dst_hbm, num_hbm, den_hbm,
               acc, as_t, ad_t, sbuf, dbuf, den_l, exs, rows0, rows1,
               scb0, scb1, gsem0, gsem1, ssem0, ssem1):
    cid = lax.axis_index("c")
    sid = lax.axis_index("s")
    w = sid * NC + cid
    bufs = (rows0, rows1)
    scbs = (scb0, scb1)
    gsems = (gsem0, gsem1)
    ssems = (ssem0, ssem1)

    pltpu.sync_copy(as_hbm.at[pl.ds(0, NT)], as_t)
    pltpu.sync_copy(ad_hbm.at[pl.ds(0, NT)], ad_t)

    zv = jnp.zeros((LANES,), jnp.float32)
    for r in range(BB):
        for j in range(D // LANES):
            rows0[r, pl.ds(j * LANES, LANES)] = zv
    for r in range(NP // LANES):
        den_l[pl.ds(r * LANES, LANES)] = zv
    nfull = RPT // BB
    tail = RPT - nfull * BB
    for c in range(nfull):
        pltpu.sync_copy(rows0, acc.at[pl.ds(sid * RPT + c * BB, BB)])
    pltpu.sync_copy(rows0.at[pl.ds(0, tail)],
                    acc.at[pl.ds(sid * RPT + nfull * BB, tail)])
    plsc.subcore_barrier()

    def chunk(c, carry):
        off = pl.multiple_of(w * WB + c * CHB, CHB)
        pltpu.sync_copy(src_hbm.at[pl.ds(off, CHB)], sbuf)
        pltpu.sync_copy(dst_hbm.at[pl.ds(off, CHB)], dbuf)

        gathers = [None, None]
        scats = [None, None]
        gathers[0] = pltpu.async_copy(h_hbm.at[sbuf.at[0]], bufs[0], gsems[0])
        for bi in range(CHB):
            par = bi % 2
            for v in range(BB // LANES):
                sl = pl.ds(v * LANES, LANES)
                s = sbuf[bi, sl]
                d = dbuf[bi, sl]
                e = plsc.load_gather(as_t, [s]) + plsc.load_gather(ad_t, [d])
                e = jnp.where(e > 0, e, 0.2 * e)
                ex = jnp.exp(e)
                exs[sl] = ex
                plsc.addupdate_scatter(den_l, [d], ex)

            if bi + 1 < CHB:
                gathers[1 - par] = pltpu.async_copy(
                    h_hbm.at[sbuf.at[bi + 1]], bufs[1 - par], gsems[1 - par])
            gathers[par].wait()
            if bi >= 2:
                scats[par].wait()

            rbuf = bufs[par]
            scbuf = scbs[par]

            @plsc.parallel_loop(0, BB, unroll=4)
            def _scale(k):
                wv = plsc.load_gather(exs, [jnp.full((LANES,), k, jnp.int32)])
                for j in range(D // LANES):
                    sl = pl.ds(j * LANES, LANES)
                    scbuf[k, sl] = rbuf[k, sl] * wv

            scats[par] = pltpu.async_copy(
                scbuf, acc.at[dbuf.at[bi]], ssems[par], add=True)
        scats[0].wait()
        scats[1].wait()
        return carry

    lax.fori_loop(0, NCH, chunk, 0)
    pltpu.sync_copy(den_l, den_hbm.at[pl.ds(w * NP, NP)])
    plsc.subcore_barrier()

    for c in range(nfull):
        base = pl.multiple_of(sid * RPT + c * BB, 8)
        pltpu.sync_copy(acc.at[pl.ds(base, BB)], rows0)
        pltpu.sync_copy(rows0, num_hbm.at[cid, pl.ds(base, BB)])
    base = pl.multiple_of(sid * RPT + nfull * BB, 8)
    pltpu.sync_copy(acc.at[pl.ds(base, tail)], rows0.at[pl.ds(0, tail)])
    pltpu.sync_copy(rows0.at[pl.ds(0, tail)], num_hbm.at[cid, pl.ds(base, tail)])


_edge_pass = pl.kernel(
    _edge_body,
    out_type=(jax.ShapeDtypeStruct((NC, NP, D), jnp.float32),
              jax.ShapeDtypeStruct((NW * NP,), jnp.float32)),
    mesh=plsc.VectorSubcoreMesh(core_axis_name="c", subcore_axis_name="s",
                                num_cores=NC, num_subcores=NS),
    compiler_params=pltpu.CompilerParams(needs_layout_passes=False),
    scratch_types=[
        pltpu.VMEM_SHARED((NT, D), jnp.float32),
        pltpu.VMEM((NT,), jnp.float32),
        pltpu.VMEM((NT,), jnp.float32),
        pltpu.VMEM((CHB, BB), jnp.int32),
        pltpu.VMEM((CHB, BB), jnp.int32),
        pltpu.VMEM((NP,), jnp.float32),
        pltpu.VMEM((BB,), jnp.float32),
        pltpu.VMEM((BB, D), jnp.float32),
        pltpu.VMEM((BB, D), jnp.float32),
        pltpu.VMEM((BB, D), jnp.float32),
        pltpu.VMEM((BB, D), jnp.float32),
        pltpu.SemaphoreType.DMA,
        pltpu.SemaphoreType.DMA,
        pltpu.SemaphoreType.DMA,
        pltpu.SemaphoreType.DMA,
    ],
)


def _ingest_body(x_ref, w_ref, asr_ref, adr_ref, h_ref, as_ref, ad_ref):
    h = jnp.dot(x_ref[...], w_ref[...], preferred_element_type=jnp.float32)
    h_ref[...] = h
    as_ref[...] = jnp.sum(h * asr_ref[...][None, :], axis=1)
    ad_ref[...] = jnp.sum(h * adr_ref[...][None, :], axis=1)


def _combine_body(num_ref, den_ref, b_ref, w_ref, asr_ref, adr_ref,
                  h_ref, as_ref, ad_ref):
    den = jnp.sum(den_ref[...], axis=0) + 1e-16
    g = (num_ref[0] + num_ref[1]) / den[:, None] + b_ref[...][None, :]
    g = jnp.maximum(g, 0.0)
    h = jnp.dot(g, w_ref[...], preferred_element_type=jnp.float32)
    h_ref[...] = h
    as_ref[...] = jnp.sum(h * asr_ref[...][None, :], axis=1)
    ad_ref[...] = jnp.sum(h * adr_ref[...][None, :], axis=1)


def _head_body(num_ref, den_ref, b_ref, w1_ref, b1_ref, w2_ref, b2_ref, y_ref):
    den = jnp.sum(den_ref[...], axis=0) + 1e-16
    g = (num_ref[0] + num_ref[1]) / den[:, None] + b_ref[...][None, :]
    g = jnp.maximum(g, 0.0)
    t = jnp.dot(g, w1_ref[...], preferred_element_type=jnp.float32)
    t = jnp.maximum(t + b1_ref[...][None, :], 0.0)
    y_ref[...] = jnp.dot(t, w2_ref[...],
                         preferred_element_type=jnp.float32) + b2_ref[...][None, :]


_vecspec = pl.BlockSpec((D,), lambda i: (0,))
_matspec = pl.BlockSpec((D, D), lambda i: (0, 0))
_rowspec = pl.BlockSpec((RB, D), lambda i: (i, 0))
_colspec = pl.BlockSpec((RB,), lambda i: (i,))
_numspec = pl.BlockSpec((NC, RB, D), lambda i: (0, i, 0))
_denspec = pl.BlockSpec((NW, RB), lambda i: (0, i))

_hao_shapes = (jax.ShapeDtypeStruct((NP, D), jnp.float32),
               jax.ShapeDtypeStruct((NP,), jnp.float32),
               jax.ShapeDtypeStruct((NP,), jnp.float32))
_hao_specs = [_rowspec, _colspec, _colspec]


def _ingest(xp, W, a_src, a_dst):
    return pl.pallas_call(
        _ingest_body,
        grid=(NP // RB,),
        in_specs=[_rowspec, _matspec, _vecspec, _vecspec],
        out_specs=_hao_specs,
        out_shape=_hao_shapes,
    )(xp, W, a_src, a_dst)


def _combine(num, den, b, W, a_src, a_dst):
    return pl.pallas_call(
        _combine_body,
        grid=(NP // RB,),
        in_specs=[_numspec, _denspec, _vecspec, _matspec, _vecspec, _vecspec],
        out_specs=_hao_specs,
        out_shape=_hao_shapes,
    )(num, den, b, W, a_src, a_dst)


def _head(num, den, b, lin1_W, lin1_b, lin2_W, lin2_b):
    return pl.pallas_call(
        _head_body,
        grid=(NP // RB,),
        in_specs=[_numspec, _denspec, _vecspec, _matspec, _vecspec,
                  _matspec, _vecspec],
        out_specs=_rowspec,
        out_shape=jax.ShapeDtypeStruct((NP, D), jnp.float32),
    )(num, den, b, lin1_W, lin1_b, lin2_W, lin2_b)


def kernel(x, edge_index, W1, a_src1, a_dst1, b1, W2, a_src2, a_dst2, b2,
           W3, a_src3, a_dst3, b3, lin1_W, lin1_b, lin2_W, lin2_b):
    src = edge_index[0]
    dst = edge_index[1]
    xp = jnp.zeros((NP, D), jnp.float32).at[:N].set(x)
    pad = EP - E0
    fill = DUM + jnp.arange(pad, dtype=jnp.int32) % (NT - N)
    srcp = jnp.concatenate([src, fill]).reshape(NW * WB, BB)
    dstp = jnp.concatenate([dst, fill]).reshape(NW * WB, BB)

    h, as_, ad = _ingest(xp, W1, a_src1, a_dst1)
    num, den = _edge_pass(h, as_, ad, srcp, dstp)
    h, as_, ad = _combine(num, den.reshape(NW, NP), b1, W2, a_src2, a_dst2)
    num, den = _edge_pass(h, as_, ad, srcp, dstp)
    h, as_, ad = _combine(num, den.reshape(NW, NP), b2, W3, a_src3, a_dst3)
    num, den = _edge_pass(h, as_, ad, srcp, dstp)
    y = _head(num, den.reshape(NW, NP), b3, lin1_W, lin1_b, lin2_W, lin2_b)
    return y[:N]


# speedup vs baseline: 1.2785x; 1.2785x over previous
import jax
import jax.numpy as jnp
from jax import lax
from jax.experimental import pallas as pl
from jax.experimental.pallas import tpu as pltpu
from jax.experimental.pallas import tpu_sc as plsc

N = 10000
D = 128
NP = 10240
NC = 2
NS = 16
LANES = 16
NW = NC * NS
BB = 64
E0 = 320000
WB = 160
CHB = 8
NCH = WB // CHB
EP = NW * WB * BB
NT = 10112
DUM = N
RPT = NT // NS
RB = 1024


def _edge_body(h_hbm, as_hbm, ad_hbm, src_hbm, dst_hbm, num_hbm, den_hbm,
               acc, as_t, ad_t, sbuf, dbuf, den_l, exs, rows0, rows1,
               gsem0, gsem1, ssem0, ssem1):
    cid = lax.axis_index("c")
    sid = lax.axis_index("s")
    w = sid * NC + cid
    bufs = (rows0, rows1)
    gsems = (gsem0, gsem1)
    ssems = (ssem0, ssem1)

    pltpu.sync_copy(as_hbm.at[pl.ds(0, NT)], as_t)
    pltpu.sync_copy(ad_hbm.at[pl.ds(0, NT)], ad_t)

    zv = jnp.zeros((LANES,), jnp.float32)
    for r in range(BB):
        for j in range(D // LANES):
            rows0[r, pl.ds(j * LANES, LANES)] = zv
    for r in range(NP // LANES):
        den_l[pl.ds(r * LANES, LANES)] = zv
    nfull = RPT // BB
    tail = RPT - nfull * BB
    for c in range(nfull):
        pltpu.sync_copy(rows0, acc.at[pl.ds(sid * RPT + c * BB, BB)])
    pltpu.sync_copy(rows0.at[pl.ds(0, tail)],
                    acc.at[pl.ds(sid * RPT + nfull * BB, tail)])
    plsc.subcore_barrier()

    def chunk(c, carry):
        off = pl.multiple_of(w * WB + c * CHB, CHB)
        pltpu.sync_copy(src_hbm.at[pl.ds(off, CHB)], sbuf)
        pltpu.sync_copy(dst_hbm.at[pl.ds(off, CHB)], dbuf)

        gathers = [None, None]
        scats = [None, None]
        gathers[0] = pltpu.async_copy(h_hbm.at[sbuf.at[0]], bufs[0], gsems[0])
        for bi in range(CHB):
            par = bi % 2
            for v in range(BB // LANES):
                sl = pl.ds(v * LANES, LANES)
                s = sbuf[bi, sl]
                d = dbuf[bi, sl]
                e = plsc.load_gather(as_t, [s]) + plsc.load_gather(ad_t, [d])
                e = jnp.where(e > 0, e, 0.2 * e)
                ex = jnp.exp(e)
                exs[sl] = ex
                plsc.addupdate_scatter(den_l, [d], ex)

            if bi + 1 < CHB:
                if bi >= 1:
                    scats[1 - par].wait()
                gathers[1 - par] = pltpu.async_copy(
                    h_hbm.at[sbuf.at[bi + 1]], bufs[1 - par], gsems[1 - par])
            gathers[par].wait()

            rbuf = bufs[par]

            @plsc.parallel_loop(0, BB, unroll=4)
            def _scale(k):
                wv = plsc.load_gather(exs, [jnp.full((LANES,), k, jnp.int32)])
                for j in range(D // LANES):
                    sl = pl.ds(j * LANES, LANES)
                    rbuf[k, sl] = rbuf[k, sl] * wv

            scats[par] = pltpu.async_copy(
                rbuf, acc.at[dbuf.at[bi]], ssems[par], add=True)
        scats[0].wait()
        scats[1].wait()
        return carry

    lax.fori_loop(0, NCH, chunk, 0)
    pltpu.sync_copy(den_l, den_hbm.at[pl.ds(w * NP, NP)])
    plsc.subcore_barrier()

    base = pl.multiple_of(sid * RPT, 8)
    pltpu.sync_copy(acc.at[pl.ds(base, RPT)],
                    num_hbm.at[cid, pl.ds(base, RPT)])


_edge_pass = pl.kernel(
    _edge_body,
    out_type=(jax.ShapeDtypeStruct((NC, NP, D), jnp.float32),
              jax.ShapeDtypeStruct((NW * NP,), jnp.float32)),
    mesh=plsc.VectorSubcoreMesh(core_axis_name="c", subcore_axis_name="s",
                                num_cores=NC, num_subcores=NS),
    compiler_params=pltpu.CompilerParams(needs_layout_passes=False),
    scratch_types=[
        pltpu.VMEM_SHARED((NT, D), jnp.float32),
        pltpu.VMEM((NT,), jnp.float32),
        pltpu.VMEM((NT,), jnp.float32),
        pltpu.VMEM((CHB, BB), jnp.int32),
        pltpu.VMEM((CHB, BB), jnp.int32),
        pltpu.VMEM((NP,), jnp.float32),
        pltpu.VMEM((BB,), jnp.float32),
        pltpu.VMEM((BB, D), jnp.float32),
        pltpu.VMEM((BB, D), jnp.float32),
        pltpu.SemaphoreType.DMA,
        pltpu.SemaphoreType.DMA,
        pltpu.SemaphoreType.DMA,
        pltpu.SemaphoreType.DMA,
    ],
)


def _ingest_body(x_ref, w_ref, asr_ref, adr_ref, h_ref, as_ref, ad_ref):
    h = jnp.dot(x_ref[...], w_ref[...], preferred_element_type=jnp.float32)
    h_ref[...] = h
    as_ref[...] = jnp.sum(h * asr_ref[...][None, :], axis=1)
    ad_ref[...] = jnp.sum(h * adr_ref[...][None, :], axis=1)


def _combine_body(num_ref, den_ref, b_ref, w_ref, asr_ref, adr_ref,
                  h_ref, as_ref, ad_ref):
    den = jnp.sum(den_ref[...], axis=0) + 1e-16
    g = (num_ref[0] + num_ref[1]) / den[:, None] + b_ref[...][None, :]
    g = jnp.maximum(g, 0.0)
    h = jnp.dot(g, w_ref[...], preferred_element_type=jnp.float32)
    h_ref[...] = h
    as_ref[...] = jnp.sum(h * asr_ref[...][None, :], axis=1)
    ad_ref[...] = jnp.sum(h * adr_ref[...][None, :], axis=1)


def _head_body(num_ref, den_ref, b_ref, w1_ref, b1_ref, w2_ref, b2_ref, y_ref):
    den = jnp.sum(den_ref[...], axis=0) + 1e-16
    g = (num_ref[0] + num_ref[1]) / den[:, None] + b_ref[...][None, :]
    g = jnp.maximum(g, 0.0)
    t = jnp.dot(g, w1_ref[...], preferred_element_type=jnp.float32)
    t = jnp.maximum(t + b1_ref[...][None, :], 0.0)
    y_ref[...] = jnp.dot(t, w2_ref[...],
                         preferred_element_type=jnp.float32) + b2_ref[...][None, :]


_vecspec = pl.BlockSpec((D,), lambda i: (0,))
_matspec = pl.BlockSpec((D, D), lambda i: (0, 0))
_rowspec = pl.BlockSpec((RB, D), lambda i: (i, 0))
_colspec = pl.BlockSpec((RB,), lambda i: (i,))
_numspec = pl.BlockSpec((NC, RB, D), lambda i: (0, i, 0))
_denspec = pl.BlockSpec((NW, RB), lambda i: (0, i))

_hao_shapes = (jax.ShapeDtypeStruct((NP, D), jnp.float32),
               jax.ShapeDtypeStruct((NP,), jnp.float32),
               jax.ShapeDtypeStruct((NP,), jnp.float32))
_hao_specs = [_rowspec, _colspec, _colspec]


def _ingest(xp, W, a_src, a_dst):
    return pl.pallas_call(
        _ingest_body,
        grid=(NP // RB,),
        in_specs=[_rowspec, _matspec, _vecspec, _vecspec],
        out_specs=_hao_specs,
        out_shape=_hao_shapes,
    )(xp, W, a_src, a_dst)


def _combine(num, den, b, W, a_src, a_dst):
    return pl.pallas_call(
        _combine_body,
        grid=(NP // RB,),
        in_specs=[_numspec, _denspec, _vecspec, _matspec, _vecspec, _vecspec],
        out_specs=_hao_specs,
        out_shape=_hao_shapes,
    )(num, den, b, W, a_src, a_dst)


def _head(num, den, b, lin1_W, lin1_b, lin2_W, lin2_b):
    return pl.pallas_call(
        _head_body,
        grid=(NP // RB,),
        in_specs=[_numspec, _denspec, _vecspec, _matspec, _vecspec,
                  _matspec, _vecspec],
        out_specs=_rowspec,
        out_shape=jax.ShapeDtypeStruct((NP, D), jnp.float32),
    )(num, den, b, lin1_W, lin1_b, lin2_W, lin2_b)


def kernel(x, edge_index, W1, a_src1, a_dst1, b1, W2, a_src2, a_dst2, b2,
           W3, a_src3, a_dst3, b3, lin1_W, lin1_b, lin2_W, lin2_b):
    src = edge_index[0]
    dst = edge_index[1]
    xp = jnp.zeros((NP, D), jnp.float32).at[:N].set(x)
    pad = EP - E0
    fill = DUM + jnp.arange(pad, dtype=jnp.int32) % (NT - N)
    srcp = jnp.concatenate([src, fill]).reshape(NW * WB, BB)
    dstp = jnp.concatenate([dst, fill]).reshape(NW * WB, BB)

    h, as_, ad = _ingest(xp, W1, a_src1, a_dst1)
    num, den = _edge_pass(h, as_, ad, srcp, dstp)
    h, as_, ad = _combine(num, den.reshape(NW, NP), b1, W2, a_src2, a_dst2)
    num, den = _edge_pass(h, as_, ad, srcp, dstp)
    h, as_, ad = _combine(num, den.reshape(NW, NP), b2, W3, a_src3, a_dst3)
    num, den = _edge_pass(h, as_, ad, srcp, dstp)
    y = _head(num, den.reshape(NW, NP), b3, lin1_W, lin1_b, lin2_W, lin2_b)
    return y[:N]
